# Initial kernel scaffold; baseline (speedup 1.0000x reference)
import functools

import jax
import jax.numpy as jnp
from jax import lax
from jax.experimental import pallas as pl
from jax.experimental.pallas import tpu as pltpu
from jax.experimental.pallas import tpu_sc as plsc

B = 16384
NEG = -1e30
NC = 2
NS = 16
NW = NC * NS
RPW = B // NW
CH = 64
NCH = RPW // CH

THICK_K, AREA_K, VOL_K, SUBVOL_K = 7, 7, 7, 2


def _scores_body(t, a, v, s,
                 tW1, tb1, tW2, tb2,
                 aW1, ab1, aW2, ab2,
                 vW1, vb1, vW2, vb2,
                 sW1, sb1, sW2, sb2,
                 ot, oa, ov, os_):
    def mlp(x, W1, b1, W2, b2):
        h = jnp.maximum(
            jnp.dot(x[...], W1[...], preferred_element_type=jnp.float32)
            + b1[...], 0.0)
        return jnp.dot(h, W2[...], preferred_element_type=jnp.float32) + b2[...]

    ot[...] = mlp(t, tW1, tb1, tW2, tb2)
    oa[...] = mlp(a, aW1, ab1, aW2, ab2)
    ov[...] = mlp(v, vW1, vb1, vW2, vb2)
    os_[...] = mlp(s, sW1, sb1, sW2, sb2)


def _tc_scores(thick, area, vol, sub_vol, tw, aw, vw, sw, R1=1024):
    grid = (B // R1,)

    def data_spec(cols):
        return pl.BlockSpec((R1, cols), lambda i: (i, 0))

    def full_spec(arr):
        return pl.BlockSpec(arr.shape, lambda i: (0,) * arr.ndim)

    in_specs = [data_spec(68), data_spec(68), data_spec(68), data_spec(16)]
    ws = list(tw) + list(aw) + list(vw) + list(sw)
    in_specs += [full_spec(w) for w in ws]
    out_specs = [data_spec(80), data_spec(80), data_spec(80), data_spec(16)]
    out_shape = [jax.ShapeDtypeStruct((B, 80), jnp.float32)] * 3 + [
        jax.ShapeDtypeStruct((B, 16), jnp.float32)]
    return pl.pallas_call(
        _scores_body, grid=grid, in_specs=in_specs, out_specs=out_specs,
        out_shape=out_shape,
    )(thick, area, vol, sub_vol, *ws)


def _merge(ka, va, kb, vb):
    rkb = jnp.flip(kb)
    rvb = jnp.flip(vb)
    c = ka >= rkb
    hk = jnp.where(c, ka, rkb)
    hv = jnp.where(c, va, rvb)
    return plsc.sort_key_val(hk, hv, descending=True)


def _topk_row(sc_ref, r, nchunk, lane):
    ks, vs = [], []
    for j in range(nchunk):
        key = sc_ref[r, pl.ds(j * 16, 16)]
        kk, vv = plsc.sort_key_val(key, lane + j * 16, descending=True)
        ks.append(kk)
        vs.append(vv)
    while len(ks) > 1:
        nk, nv = [], []
        for i in range(0, len(ks) - 1, 2):
            kk, vv = _merge(ks[i], vs[i], ks[i + 1], vs[i + 1])
            nk.append(kk)
            nv.append(vv)
        if len(ks) % 2:
            nk.append(ks[-1])
            nv.append(vs[-1])
        ks, vs = nk, nv
    return ks[0], vs[0]


def _branch_row(sc_ref, x_ref, r, nchunk, mk, lane,
                feat_v, feat_off, idx_v, w_v, fm):
    keys, vals = _topk_row(sc_ref, r, nchunk, lane)
    mx = jnp.max(keys)
    e = jnp.where(mk, jnp.exp(keys - mx), 0.0)
    w = e / jnp.sum(e)
    idx = jnp.where(mk, vals, 0)
    rvec = jnp.full((16,), r, jnp.int32)
    xs = plsc.load_gather(x_ref, [rvec, idx], mask=mk)
    wt = jnp.where(mk, xs * w, 0.0)
    plsc.store_scatter(idx_v, [rvec, lane], idx, mask=mk)
    plsc.store_scatter(w_v, [rvec, lane], w, mask=mk)
    plsc.store_scatter(feat_v, [rvec, lane + feat_off], wt, mask=fm)


def _sc_body(t_h, a_h, v_h, s_h, sct_h, sca_h, scv_h, scs_h,
             feat_h, ti_h, tw_h, ai_h, aw_h, vi_h, vw_h, si_h, sw_h,
             xt, xa, xv, xs, sct, sca, scv, scs,
             feat, ti, tw, ai, aw, vi, vw, si, sw):
    wid = lax.axis_index("s") * NC + lax.axis_index("c")
    lane = lax.iota(jnp.int32, 16)
    m7 = lane < 7
    m2 = lane < 2
    m3 = lane < 3

    def chunk_body(c, _):
        base = wid * RPW + c * CH
        pltpu.sync_copy(t_h.at[pl.ds(base, CH)], xt)
        pltpu.sync_copy(a_h.at[pl.ds(base, CH)], xa)
        pltpu.sync_copy(v_h.at[pl.ds(base, CH)], xv)
        pltpu.sync_copy(s_h.at[pl.ds(base, CH)], xs)
        pltpu.sync_copy(sct_h.at[pl.ds(base, CH)], sct)
        pltpu.sync_copy(sca_h.at[pl.ds(base, CH)], sca)
        pltpu.sync_copy(scv_h.at[pl.ds(base, CH)], scv)
        pltpu.sync_copy(scs_h.at[pl.ds(base, CH)], scs)

        def row_body(r, __):
            _branch_row(sct, xt, r, 5, m7, lane, feat, 0, ti, tw, m7)
            _branch_row(sca, xa, r, 5, m7, lane, feat, 7, ai, aw, m7)
            _branch_row(scv, xv, r, 5, m7, lane, feat, 14, vi, vw, m7)
            _branch_row(scs, xs, r, 1, m2, lane, feat, 21, si, sw, m3)
            return 0

        lax.fori_loop(0, CH, row_body, 0)
        pltpu.sync_copy(feat, feat_h.at[pl.ds(base, CH)])
        pltpu.sync_copy(ti, ti_h.at[pl.ds(base, CH)])
        pltpu.sync_copy(tw, tw_h.at[pl.ds(base, CH)])
        pltpu.sync_copy(ai, ai_h.at[pl.ds(base, CH)])
        pltpu.sync_copy(aw, aw_h.at[pl.ds(base, CH)])
        pltpu.sync_copy(vi, vi_h.at[pl.ds(base, CH)])
        pltpu.sync_copy(vw, vw_h.at[pl.ds(base, CH)])
        pltpu.sync_copy(si, si_h.at[pl.ds(base, CH)])
        pltpu.sync_copy(sw, sw_h.at[pl.ds(base, CH)])
        return 0

    lax.fori_loop(0, NCH, chunk_body, 0)


def _sc_topk(thick, area, vol, sub_vol, sct, sca, scv, scs):
    f32, i32 = jnp.float32, jnp.int32
    mesh = plsc.VectorSubcoreMesh(core_axis_name="c", subcore_axis_name="s")
    out_type = [
        jax.ShapeDtypeStruct((B, 24), f32),
        jax.ShapeDtypeStruct((B, 7), i32), jax.ShapeDtypeStruct((B, 7), f32),
        jax.ShapeDtypeStruct((B, 7), i32), jax.ShapeDtypeStruct((B, 7), f32),
        jax.ShapeDtypeStruct((B, 7), i32), jax.ShapeDtypeStruct((B, 7), f32),
        jax.ShapeDtypeStruct((B, 2), i32), jax.ShapeDtypeStruct((B, 2), f32),
    ]
    scratch = [
        pltpu.VMEM((CH, 68), f32), pltpu.VMEM((CH, 68), f32),
        pltpu.VMEM((CH, 68), f32), pltpu.VMEM((CH, 16), f32),
        pltpu.VMEM((CH, 80), f32), pltpu.VMEM((CH, 80), f32),
        pltpu.VMEM((CH, 80), f32), pltpu.VMEM((CH, 16), f32),
        pltpu.VMEM((CH, 24), f32),
        pltpu.VMEM((CH, 7), i32), pltpu.VMEM((CH, 7), f32),
        pltpu.VMEM((CH, 7), i32), pltpu.VMEM((CH, 7), f32),
        pltpu.VMEM((CH, 7), i32), pltpu.VMEM((CH, 7), f32),
        pltpu.VMEM((CH, 2), i32), pltpu.VMEM((CH, 2), f32),
    ]
    fn = pl.kernel(_sc_body, out_type=out_type, mesh=mesh,
                   scratch_types=scratch)
    return fn(thick, area, vol, sub_vol, sct, sca, scv, scs)


def _final_body(feat, f1W, f1b, f2w, f2b, x1_ref, out_ref):
    x1 = jnp.dot(feat[...], f1W[...],
                 preferred_element_type=jnp.float32) + f1b[...]
    x1_ref[...] = x1
    xr = jnp.maximum(x1, 0.0)
    out_ref[...] = jnp.sum(xr * f2w[...], axis=1, keepdims=True) + f2b[...]


def _tc_final(feat, f1Wp, f1b, f2w, f2b, R3=1024):
    grid = (B // R3,)

    def full_spec(arr):
        return pl.BlockSpec(arr.shape, lambda i: (0,) * arr.ndim)

    return pl.pallas_call(
        _final_body, grid=grid,
        in_specs=[pl.BlockSpec((R3, 24), lambda i: (i, 0)),
                  full_spec(f1Wp), full_spec(f1b), full_spec(f2w),
                  full_spec(f2b)],
        out_specs=[pl.BlockSpec((R3, 256), lambda i: (i, 0)),
                   pl.BlockSpec((R3, 1), lambda i: (i, 0))],
        out_shape=[jax.ShapeDtypeStruct((B, 256), jnp.float32),
                   jax.ShapeDtypeStruct((B, 1), jnp.float32)],
    )(feat, f1Wp, f1b, f2w, f2b)


def kernel(thick, area, vol, sub_vol,
           tW1, tb1, tW2, tb2, aW1, ab1, aW2, ab2,
           vW1, vb1, vW2, vb2, sW1, sb1, sW2, sb2,
           f1W, f1b, f2W, f2b):
    f32 = jnp.float32
    pad12 = jnp.full((12,), NEG, f32)

    def padw(W2, b2):
        return (jnp.pad(W2, ((0, 0), (0, 12))),
                jnp.concatenate([b2, pad12]).reshape(1, 80))

    tW2p, tb2p = padw(tW2, tb2)
    aW2p, ab2p = padw(aW2, ab2)
    vW2p, vb2p = padw(vW2, vb2)

    tw = (tW1, tb1.reshape(1, 32), tW2p, tb2p)
    aw = (aW1, ab1.reshape(1, 32), aW2p, ab2p)
    vw = (vW1, vb1.reshape(1, 32), vW2p, vb2p)
    sw = (sW1, sb1.reshape(1, 8), sW2, sb2.reshape(1, 16))

    sct, sca, scv, scs = _tc_scores(thick, area, vol, sub_vol, tw, aw, vw, sw)

    (feat, t_idx, t_w, a_idx, a_w, v_idx, v_w, s_idx, s_w) = _sc_topk(
        thick, area, vol, sub_vol, sct, sca, scv, scs)

    f1Wp = jnp.pad(f1W, ((0, 1), (0, 0)))
    x1, out = _tc_final(feat, f1Wp, f1b.reshape(1, 256),
                        f2W.reshape(1, 256), f2b.reshape(1, 1))

    return (out, x1, t_idx, t_w, a_idx, a_w, v_idx, v_w, s_idx, s_w)


# speedup vs baseline: 1.4456x; 1.4456x over previous
import functools

import jax
import jax.numpy as jnp
from jax import lax
from jax.experimental import pallas as pl
from jax.experimental.pallas import tpu as pltpu
from jax.experimental.pallas import tpu_sc as plsc

B = 16384
NEG = -1e30
NC = 2
NS = 16
NW = NC * NS
RPW = B // NW
CH = 64
NCH = RPW // CH

THICK_K, AREA_K, VOL_K, SUBVOL_K = 7, 7, 7, 2


def _scores_body(t, a, v, s,
                 tW1, tb1, tW2, tb2,
                 aW1, ab1, aW2, ab2,
                 vW1, vb1, vW2, vb2,
                 sW1, sb1, sW2, sb2,
                 ot, oa, ov, os_):
    def mlp(x, W1, b1, W2, b2):
        h = jnp.maximum(
            jnp.dot(x[...], W1[...], preferred_element_type=jnp.float32)
            + b1[...], 0.0)
        return jnp.dot(h, W2[...], preferred_element_type=jnp.float32) + b2[...]

    ot[...] = mlp(t, tW1, tb1, tW2, tb2)
    oa[...] = mlp(a, aW1, ab1, aW2, ab2)
    ov[...] = mlp(v, vW1, vb1, vW2, vb2)
    os_[...] = mlp(s, sW1, sb1, sW2, sb2)


def _tc_scores(thick, area, vol, sub_vol, tw, aw, vw, sw, R1=1024):
    grid = (B // R1,)

    def data_spec(cols):
        return pl.BlockSpec((R1, cols), lambda i: (i, 0))

    def full_spec(arr):
        return pl.BlockSpec(arr.shape, lambda i: (0,) * arr.ndim)

    in_specs = [data_spec(68), data_spec(68), data_spec(68), data_spec(16)]
    ws = list(tw) + list(aw) + list(vw) + list(sw)
    in_specs += [full_spec(w) for w in ws]
    out_specs = [data_spec(80), data_spec(80), data_spec(80), data_spec(16)]
    out_shape = [jax.ShapeDtypeStruct((B, 80), jnp.float32)] * 3 + [
        jax.ShapeDtypeStruct((B, 16), jnp.float32)]
    return pl.pallas_call(
        _scores_body, grid=grid, in_specs=in_specs, out_specs=out_specs,
        out_shape=out_shape,
    )(thick, area, vol, sub_vol, *ws)


def _merge(ka, va, kb, vb):
    rkb = jnp.flip(kb)
    rvb = jnp.flip(vb)
    c = ka >= rkb
    hk = jnp.where(c, ka, rkb)
    hv = jnp.where(c, va, rvb)
    return plsc.sort_key_val(hk, hv, descending=True)


def _topk_row(sc_ref, r, nchunk, lane):
    ks, vs = [], []
    for j in range(nchunk):
        key = sc_ref[r, pl.ds(j * 16, 16)]
        kk, vv = plsc.sort_key_val(key, lane + j * 16, descending=True)
        ks.append(kk)
        vs.append(vv)
    while len(ks) > 1:
        nk, nv = [], []
        for i in range(0, len(ks) - 1, 2):
            kk, vv = _merge(ks[i], vs[i], ks[i + 1], vs[i + 1])
            nk.append(kk)
            nv.append(vv)
        if len(ks) % 2:
            nk.append(ks[-1])
            nv.append(vs[-1])
        ks, vs = nk, nv
    return ks[0], vs[0]


def _branch_row(sc_ref, x_ref, r, nchunk, mk, lane,
                feat_v, feat_off, idx_v, w_v, fm):
    keys, vals = _topk_row(sc_ref, r, nchunk, lane)
    mx = jnp.max(keys)
    e = jnp.where(mk, jnp.exp(keys - mx), 0.0)
    w = e / jnp.sum(e)
    idx = jnp.where(mk, vals, 0)
    rvec = jnp.full((16,), r, jnp.int32)
    xs = plsc.load_gather(x_ref, [rvec, idx], mask=mk)
    wt = jnp.where(mk, xs * w, 0.0)
    plsc.store_scatter(idx_v, [rvec, lane], idx, mask=mk)
    plsc.store_scatter(w_v, [rvec, lane], w, mask=mk)
    plsc.store_scatter(feat_v, [rvec, lane + feat_off], wt, mask=fm)


def _sc_body(t_h, a_h, v_h, s_h, sct_h, sca_h, scv_h, scs_h,
             feat_h, ti_h, tw_h, ai_h, aw_h, vi_h, vw_h, si_h, sw_h,
             xt, xa, xv, xs, sct, sca, scv, scs,
             feat, ti, tw, ai, aw, vi, vw, si, sw):
    wid = lax.axis_index("s") * NC + lax.axis_index("c")
    lane = lax.iota(jnp.int32, 16)
    m7 = lane < 7
    m2 = lane < 2
    m3 = lane < 3

    def chunk_body(c, _):
        base = wid * RPW + c * CH
        pltpu.sync_copy(t_h.at[pl.ds(base, CH)], xt)
        pltpu.sync_copy(a_h.at[pl.ds(base, CH)], xa)
        pltpu.sync_copy(v_h.at[pl.ds(base, CH)], xv)
        pltpu.sync_copy(s_h.at[pl.ds(base, CH)], xs)
        pltpu.sync_copy(sct_h.at[pl.ds(base, CH)], sct)
        pltpu.sync_copy(sca_h.at[pl.ds(base, CH)], sca)
        pltpu.sync_copy(scv_h.at[pl.ds(base, CH)], scv)
        pltpu.sync_copy(scs_h.at[pl.ds(base, CH)], scs)

        def row_body(r, __):
            _branch_row(sct, xt, r, 5, m7, lane, feat, 0, ti, tw, m7)
            _branch_row(sca, xa, r, 5, m7, lane, feat, 7, ai, aw, m7)
            _branch_row(scv, xv, r, 5, m7, lane, feat, 14, vi, vw, m7)
            _branch_row(scs, xs, r, 1, m2, lane, feat, 21, si, sw, m3)
            return 0

        lax.fori_loop(0, CH, row_body, 0)
        pltpu.sync_copy(feat, feat_h.at[pl.ds(base, CH)])
        pltpu.sync_copy(ti, ti_h.at[pl.ds(base, CH)])
        pltpu.sync_copy(tw, tw_h.at[pl.ds(base, CH)])
        pltpu.sync_copy(ai, ai_h.at[pl.ds(base, CH)])
        pltpu.sync_copy(aw, aw_h.at[pl.ds(base, CH)])
        pltpu.sync_copy(vi, vi_h.at[pl.ds(base, CH)])
        pltpu.sync_copy(vw, vw_h.at[pl.ds(base, CH)])
        pltpu.sync_copy(si, si_h.at[pl.ds(base, CH)])
        pltpu.sync_copy(sw, sw_h.at[pl.ds(base, CH)])
        return 0

    lax.fori_loop(0, NCH, chunk_body, 0)


def _sc_topk(thick, area, vol, sub_vol, sct, sca, scv, scs):
    f32, i32 = jnp.float32, jnp.int32
    mesh = plsc.VectorSubcoreMesh(core_axis_name="c", subcore_axis_name="s")
    out_type = [
        jax.ShapeDtypeStruct((B, 24), f32),
        jax.ShapeDtypeStruct((B, 7), i32), jax.ShapeDtypeStruct((B, 7), f32),
        jax.ShapeDtypeStruct((B, 7), i32), jax.ShapeDtypeStruct((B, 7), f32),
        jax.ShapeDtypeStruct((B, 7), i32), jax.ShapeDtypeStruct((B, 7), f32),
        jax.ShapeDtypeStruct((B, 2), i32), jax.ShapeDtypeStruct((B, 2), f32),
    ]
    scratch = [
        pltpu.VMEM((CH, 68), f32), pltpu.VMEM((CH, 68), f32),
        pltpu.VMEM((CH, 68), f32), pltpu.VMEM((CH, 16), f32),
        pltpu.VMEM((CH, 80), f32), pltpu.VMEM((CH, 80), f32),
        pltpu.VMEM((CH, 80), f32), pltpu.VMEM((CH, 16), f32),
        pltpu.VMEM((CH, 24), f32),
        pltpu.VMEM((CH, 7), i32), pltpu.VMEM((CH, 7), f32),
        pltpu.VMEM((CH, 7), i32), pltpu.VMEM((CH, 7), f32),
        pltpu.VMEM((CH, 7), i32), pltpu.VMEM((CH, 7), f32),
        pltpu.VMEM((CH, 2), i32), pltpu.VMEM((CH, 2), f32),
    ]
    fn = pl.kernel(_sc_body, out_type=out_type, mesh=mesh,
                   scratch_types=scratch,
                   compiler_params=pltpu.CompilerParams(
                       needs_layout_passes=False,
                       use_tc_tiling_on_sc=False))
    return fn(thick, area, vol, sub_vol, sct, sca, scv, scs)


def _final_body(feat, f1W, f1b, f2w, f2b, x1_ref, out_ref):
    x1 = jnp.dot(feat[...], f1W[...],
                 preferred_element_type=jnp.float32) + f1b[...]
    x1_ref[...] = x1
    xr = jnp.maximum(x1, 0.0)
    out_ref[...] = jnp.sum(xr * f2w[...], axis=1, keepdims=True) + f2b[...]


def _tc_final(feat, f1Wp, f1b, f2w, f2b, R3=1024):
    grid = (B // R3,)

    def full_spec(arr):
        return pl.BlockSpec(arr.shape, lambda i: (0,) * arr.ndim)

    return pl.pallas_call(
        _final_body, grid=grid,
        in_specs=[pl.BlockSpec((R3, 24), lambda i: (i, 0)),
                  full_spec(f1Wp), full_spec(f1b), full_spec(f2w),
                  full_spec(f2b)],
        out_specs=[pl.BlockSpec((R3, 256), lambda i: (i, 0)),
                   pl.BlockSpec((R3, 1), lambda i: (i, 0))],
        out_shape=[jax.ShapeDtypeStruct((B, 256), jnp.float32),
                   jax.ShapeDtypeStruct((B, 1), jnp.float32)],
    )(feat, f1Wp, f1b, f2w, f2b)


def kernel(thick, area, vol, sub_vol,
           tW1, tb1, tW2, tb2, aW1, ab1, aW2, ab2,
           vW1, vb1, vW2, vb2, sW1, sb1, sW2, sb2,
           f1W, f1b, f2W, f2b):
    f32 = jnp.float32
    pad12 = jnp.full((12,), NEG, f32)

    def padw(W2, b2):
        return (jnp.pad(W2, ((0, 0), (0, 12))),
                jnp.concatenate([b2, pad12]).reshape(1, 80))

    tW2p, tb2p = padw(tW2, tb2)
    aW2p, ab2p = padw(aW2, ab2)
    vW2p, vb2p = padw(vW2, vb2)

    tw = (tW1, tb1.reshape(1, 32), tW2p, tb2p)
    aw = (aW1, ab1.reshape(1, 32), aW2p, ab2p)
    vw = (vW1, vb1.reshape(1, 32), vW2p, vb2p)
    sw = (sW1, sb1.reshape(1, 8), sW2, sb2.reshape(1, 16))

    sct, sca, scv, scs = _tc_scores(thick, area, vol, sub_vol, tw, aw, vw, sw)

    (feat, t_idx, t_w, a_idx, a_w, v_idx, v_w, s_idx, s_w) = _sc_topk(
        thick, area, vol, sub_vol, sct, sca, scv, scs)

    f1Wp = jnp.pad(f1W, ((0, 1), (0, 0)))
    x1, out = _tc_final(feat, f1Wp, f1b.reshape(1, 256),
                        f2W.reshape(1, 256), f2b.reshape(1, 1))

    return (out, x1, t_idx, t_w, a_idx, a_w, v_idx, v_w, s_idx, s_w)


# speedup vs baseline: 1.9738x; 1.3654x over previous
import functools

import jax
import jax.numpy as jnp
from jax import lax
from jax.experimental import pallas as pl
from jax.experimental.pallas import tpu as pltpu
from jax.experimental.pallas import tpu_sc as plsc

B = 16384
NEG = -1e30
NC = 2
NS = 16
NW = NC * NS
RPW = B // NW
CH = 64
NCH = RPW // CH

PIN = 640
SC_T, SC_A, SC_V, SC_S = 0, 128, 256, 384
X_T, X_A, X_V, X_S = 400, 468, 536, 604

POUT = 128
F_T, F_A, F_V, F_S = 0, 7, 14, 21
I_T, I_A, I_V, I_S = 24, 31, 38, 45
W_T, W_A, W_V, W_S = 47, 54, 61, 68


def _scores_body(t, a, v, s,
                 tW1, tb1, tW2, tb2,
                 aW1, ab1, aW2, ab2,
                 vW1, vb1, vW2, vb2,
                 sW1, sb1, sW2, sb2,
                 o):
    def mlp(x, W1, b1, W2, b2):
        h = jnp.maximum(
            jnp.dot(x, W1[...], preferred_element_type=jnp.float32)
            + b1[...], 0.0)
        return jnp.dot(h, W2[...], preferred_element_type=jnp.float32) + b2[...]

    tv, av, vv, sv = t[...], a[...], v[...], s[...]
    o[:, SC_T:SC_T + 128] = mlp(tv, tW1, tb1, tW2, tb2)
    o[:, SC_A:SC_A + 128] = mlp(av, aW1, ab1, aW2, ab2)
    o[:, SC_V:SC_V + 128] = mlp(vv, vW1, vb1, vW2, vb2)
    o[:, SC_S:SC_S + 16] = mlp(sv, sW1, sb1, sW2, sb2)
    o[:, X_T:X_T + 68] = tv
    o[:, X_A:X_A + 68] = av
    o[:, X_V:X_V + 68] = vv
    o[:, X_S:X_S + 16] = sv


def _tc_scores(thick, area, vol, sub_vol, tw, aw, vw, sw, R1=1024):
    grid = (B // R1,)

    def data_spec(cols):
        return pl.BlockSpec((R1, cols), lambda i: (i, 0))

    def full_spec(arr):
        return pl.BlockSpec(arr.shape, lambda i: (0,) * arr.ndim)

    in_specs = [data_spec(68), data_spec(68), data_spec(68), data_spec(16)]
    ws = list(tw) + list(aw) + list(vw) + list(sw)
    in_specs += [full_spec(w) for w in ws]
    return pl.pallas_call(
        _scores_body, grid=grid, in_specs=in_specs,
        out_specs=data_spec(PIN),
        out_shape=jax.ShapeDtypeStruct((B, PIN), jnp.float32),
    )(thick, area, vol, sub_vol, *ws)


def _merge(ka, va, kb, vb):
    rkb = jnp.flip(kb)
    rvb = jnp.flip(vb)
    c = ka >= rkb
    hk = jnp.where(c, ka, rkb)
    hv = jnp.where(c, va, rvb)
    return plsc.sort_key_val(hk, hv, descending=True)


def _topk_row(pk, r, sc_off, nchunk, lane):
    ks, vs = [], []
    for j in range(nchunk):
        key = pk[r, pl.ds(sc_off + j * 16, 16)]
        kk, vv = plsc.sort_key_val(key, lane + j * 16, descending=True)
        ks.append(kk)
        vs.append(vv)
    while len(ks) > 1:
        nk, nv = [], []
        for i in range(0, len(ks) - 1, 2):
            kk, vv = _merge(ks[i], vs[i], ks[i + 1], vs[i + 1])
            nk.append(kk)
            nv.append(vv)
        if len(ks) % 2:
            nk.append(ks[-1])
            nv.append(vs[-1])
        ks, vs = nk, nv
    return ks[0], vs[0]


def _branch_row(pk, po, r, sc_off, nchunk, x_off, mk, lane,
                f_off, i_off, w_off, fm):
    keys, vals = _topk_row(pk, r, sc_off, nchunk, lane)
    mx = jnp.max(keys)
    e = jnp.where(mk, jnp.exp(keys - mx), 0.0)
    w = e / jnp.sum(e)
    idx = jnp.where(mk, vals, 0)
    rvec = jnp.full((16,), r, jnp.int32)
    xs = plsc.load_gather(pk, [rvec, x_off + idx], mask=mk)
    wt = jnp.where(mk, xs * w, 0.0)
    plsc.store_scatter(po, [rvec, lane + i_off],
                       plsc.bitcast(idx, jnp.float32), mask=mk)
    plsc.store_scatter(po, [rvec, lane + w_off], w, mask=mk)
    plsc.store_scatter(po, [rvec, lane + f_off], wt, mask=fm)


def _sc_body(pin_h, pout_h, pk0, pk1, po0, po1, si0, si1, so0, so1):
    wid = lax.axis_index("s") * NC + lax.axis_index("c")
    lane = lax.iota(jnp.int32, 16)
    m7 = lane < 7
    m2 = lane < 2
    m3 = lane < 3
    base0 = wid * RPW

    pks = (pk0, pk1)
    pos = (po0, po1)
    sis = (si0, si1)
    sos = (so0, so1)

    def compute_chunk(pk, po):
        def row_body(r, _):
            _branch_row(pk, po, r, SC_T, 5, X_T, m7, lane, F_T, I_T, W_T, m7)
            _branch_row(pk, po, r, SC_A, 5, X_A, m7, lane, F_A, I_A, W_A, m7)
            _branch_row(pk, po, r, SC_V, 5, X_V, m7, lane, F_V, I_V, W_V, m7)
            _branch_row(pk, po, r, SC_S, 1, X_S, m2, lane, F_S, I_S, W_S, m3)
            return 0

        lax.fori_loop(0, CH, row_body, 0)

    hin = [None] * NCH
    hout = [None] * NCH
    hin[0] = pltpu.async_copy(pin_h.at[pl.ds(base0, CH)], pk0, si0)
    for c in range(NCH):
        cur = c & 1
        if c + 1 < NCH:
            hin[c + 1] = pltpu.async_copy(
                pin_h.at[pl.ds(base0 + (c + 1) * CH, CH)],
                pks[1 - cur], sis[1 - cur])
        hin[c].wait()
        if c >= 2:
            hout[c - 2].wait()
        compute_chunk(pks[cur], pos[cur])
        hout[c] = pltpu.async_copy(
            pos[cur], pout_h.at[pl.ds(base0 + c * CH, CH)], sos[cur])
    hout[NCH - 2].wait()
    hout[NCH - 1].wait()


def _sc_topk(pin):
    f32 = jnp.float32
    mesh = plsc.VectorSubcoreMesh(core_axis_name="c", subcore_axis_name="s")
    fn = pl.kernel(
        _sc_body,
        out_type=jax.ShapeDtypeStruct((B, POUT), f32),
        mesh=mesh,
        scratch_types=[
            pltpu.VMEM((CH, PIN), f32), pltpu.VMEM((CH, PIN), f32),
            pltpu.VMEM((CH, POUT), f32), pltpu.VMEM((CH, POUT), f32),
            pltpu.SemaphoreType.DMA, pltpu.SemaphoreType.DMA,
            pltpu.SemaphoreType.DMA, pltpu.SemaphoreType.DMA,
        ],
        compiler_params=pltpu.CompilerParams(
            needs_layout_passes=False,
            use_tc_tiling_on_sc=False))
    return fn(pin)


def _final_body(p, f1W, f1b, f2w, f2b,
                out_ref, x1_ref, ti, tw, ai, aw, vi, vw, si, sw):
    pv = p[...]
    feat = pv[:, 0:24]
    x1 = jnp.dot(feat, f1W[...], preferred_element_type=jnp.float32) + f1b[...]
    x1_ref[...] = x1
    xr = jnp.maximum(x1, 0.0)
    out_ref[...] = jnp.sum(xr * f2w[...], axis=1, keepdims=True) + f2b[...]
    ti[...] = lax.bitcast_convert_type(pv[:, I_T:I_T + 7], jnp.int32)
    ai[...] = lax.bitcast_convert_type(pv[:, I_A:I_A + 7], jnp.int32)
    vi[...] = lax.bitcast_convert_type(pv[:, I_V:I_V + 7], jnp.int32)
    si[...] = lax.bitcast_convert_type(pv[:, I_S:I_S + 2], jnp.int32)
    tw[...] = pv[:, W_T:W_T + 7]
    aw[...] = pv[:, W_A:W_A + 7]
    vw[...] = pv[:, W_V:W_V + 7]
    sw[...] = pv[:, W_S:W_S + 2]


def _tc_final(pout, f1Wp, f1b, f2w, f2b, R3=1024):
    grid = (B // R3,)
    f32, i32 = jnp.float32, jnp.int32

    def data_spec(cols):
        return pl.BlockSpec((R3, cols), lambda i: (i, 0))

    def full_spec(arr):
        return pl.BlockSpec(arr.shape, lambda i: (0,) * arr.ndim)

    return pl.pallas_call(
        _final_body, grid=grid,
        in_specs=[data_spec(POUT), full_spec(f1Wp), full_spec(f1b),
                  full_spec(f2w), full_spec(f2b)],
        out_specs=[data_spec(1), data_spec(256),
                   data_spec(7), data_spec(7), data_spec(7), data_spec(7),
                   data_spec(7), data_spec(7), data_spec(2), data_spec(2)],
        out_shape=[jax.ShapeDtypeStruct((B, 1), f32),
                   jax.ShapeDtypeStruct((B, 256), f32),
                   jax.ShapeDtypeStruct((B, 7), i32),
                   jax.ShapeDtypeStruct((B, 7), f32),
                   jax.ShapeDtypeStruct((B, 7), i32),
                   jax.ShapeDtypeStruct((B, 7), f32),
                   jax.ShapeDtypeStruct((B, 7), i32),
                   jax.ShapeDtypeStruct((B, 7), f32),
                   jax.ShapeDtypeStruct((B, 2), i32),
                   jax.ShapeDtypeStruct((B, 2), f32)],
    )(pout, f1Wp, f1b, f2w, f2b)


def kernel(thick, area, vol, sub_vol,
           tW1, tb1, tW2, tb2, aW1, ab1, aW2, ab2,
           vW1, vb1, vW2, vb2, sW1, sb1, sW2, sb2,
           f1W, f1b, f2W, f2b):
    f32 = jnp.float32
    pad60 = jnp.full((60,), NEG, f32)

    def padw(W2, b2):
        return (jnp.pad(W2, ((0, 0), (0, 60))),
                jnp.concatenate([b2, pad60]).reshape(1, 128))

    tW2p, tb2p = padw(tW2, tb2)
    aW2p, ab2p = padw(aW2, ab2)
    vW2p, vb2p = padw(vW2, vb2)

    tw = (tW1, tb1.reshape(1, 32), tW2p, tb2p)
    aw = (aW1, ab1.reshape(1, 32), aW2p, ab2p)
    vw = (vW1, vb1.reshape(1, 32), vW2p, vb2p)
    sw = (sW1, sb1.reshape(1, 8), sW2, sb2.reshape(1, 16))

    pin = _tc_scores(thick, area, vol, sub_vol, tw, aw, vw, sw)
    pout = _sc_topk(pin)

    f1Wp = jnp.pad(f1W, ((0, 1), (0, 0)))
    out, x1, t_idx, t_w, a_idx, a_w, v_idx, v_w, s_idx, s_w = _tc_final(
        pout, f1Wp, f1b.reshape(1, 256), f2W.reshape(1, 256),
        f2b.reshape(1, 1))

    return (out, x1, t_idx, t_w, a_idx, a_w, v_idx, v_w, s_idx, s_w)


# speedup vs baseline: 4.2677x; 2.1622x over previous
import functools

import jax
import jax.numpy as jnp
from jax import lax
from jax.experimental import pallas as pl
from jax.experimental.pallas import tpu as pltpu
from jax.experimental.pallas import tpu_sc as plsc

B = 16384
NEG = -1e30
NC = 2
NS = 16
NW = NC * NS
RPW = B // NW
CH = 64
NCH = RPW // CH
NSEC = 6

POUT = 128
F_T, F_A, F_V, F_S = 0, 7, 14, 21
I_T, I_A, I_V, I_S = 24, 31, 38, 45
W_T, W_A, W_V, W_S = 47, 54, 61, 68


def _scores_body(tT, aT, vT, sT,
                 tW1, tb1, tW2, tb2,
                 aW1, ab1, aW2, ab2,
                 vW1, vb1, vW2, vb2,
                 sW1, sb1, sW2, sb2,
                 o):
    def mlp_t(xT, W1t, b1c, W2t, b2c):
        h = jnp.maximum(
            jnp.dot(W1t[...], xT, preferred_element_type=jnp.float32)
            + b1c[...], 0.0)
        return jnp.dot(W2t[...], h, preferred_element_type=jnp.float32) + b2c[...]

    tv, av, vv, sv = tT[...], aT[...], vT[...], sT[...]
    R = RPW
    o[0:R, :] = jnp.transpose(mlp_t(tv, tW1, tb1, tW2, tb2))
    o[R:2 * R, :] = jnp.transpose(mlp_t(av, aW1, ab1, aW2, ab2))
    o[2 * R:3 * R, :] = jnp.transpose(mlp_t(vv, vW1, vb1, vW2, vb2))
    o[3 * R:4 * R, 0:16] = jnp.transpose(mlp_t(sv, sW1, sb1, sW2, sb2))
    o[3 * R:4 * R, 16:84] = jnp.transpose(tv)
    o[3 * R:4 * R, 84:100] = jnp.transpose(sv)
    o[4 * R:5 * R, 0:68] = jnp.transpose(av)
    o[5 * R:6 * R, 0:68] = jnp.transpose(vv)


def _tc_scores(tT, aT, vT, sT, tw, aw, vw, sw):
    grid = (NW,)

    def dataT_spec(rows):
        return pl.BlockSpec((rows, RPW), lambda i: (0, i))

    def full_spec(arr):
        return pl.BlockSpec(arr.shape, lambda i: (0,) * arr.ndim)

    in_specs = [dataT_spec(68), dataT_spec(68), dataT_spec(68), dataT_spec(16)]
    ws = list(tw) + list(aw) + list(vw) + list(sw)
    in_specs += [full_spec(w) for w in ws]
    return pl.pallas_call(
        _scores_body, grid=grid, in_specs=in_specs,
        out_specs=pl.BlockSpec((NSEC * RPW, POUT), lambda i: (i, 0)),
        out_shape=jax.ShapeDtypeStruct((NSEC * B, POUT), jnp.float32),
    )(tT, aT, vT, sT, *ws)


def _merge(ka, va, kb, vb):
    rkb = jnp.flip(kb)
    rvb = jnp.flip(vb)
    c = ka >= rkb
    hk = jnp.where(c, ka, rkb)
    hv = jnp.where(c, va, rvb)
    return plsc.sort_key_val(hk, hv, descending=True)


def _topk_row(sc_ref, r, nchunk, lane):
    ks, vs = [], []
    for j in range(nchunk):
        key = sc_ref[r, pl.ds(j * 16, 16)]
        kk, vv = plsc.sort_key_val(key, lane + j * 16, descending=True)
        ks.append(kk)
        vs.append(vv)
    while len(ks) > 1:
        nk, nv = [], []
        for i in range(0, len(ks) - 1, 2):
            kk, vv = _merge(ks[i], vs[i], ks[i + 1], vs[i + 1])
            nk.append(kk)
            nv.append(vv)
        if len(ks) % 2:
            nk.append(ks[-1])
            nv.append(vs[-1])
        ks, vs = nk, nv
    return ks[0], vs[0]


def _branch_row(sc_ref, x_ref, x_off, po, r, nchunk, mk, lane,
                f_off, i_off, w_off, fm):
    keys, vals = _topk_row(sc_ref, r, nchunk, lane)
    mx = jnp.max(keys)
    e = jnp.where(mk, jnp.exp(keys - mx), 0.0)
    w = e / jnp.sum(e)
    idx = jnp.where(mk, vals, 0)
    rvec = jnp.full((16,), r, jnp.int32)
    xs = plsc.load_gather(x_ref, [rvec, x_off + idx], mask=mk)
    wt = jnp.where(mk, xs * w, 0.0)
    plsc.store_scatter(po, [rvec, lane + i_off],
                       plsc.bitcast(idx, jnp.float32), mask=mk)
    plsc.store_scatter(po, [rvec, lane + w_off], w, mask=mk)
    plsc.store_scatter(po, [rvec, lane + f_off], wt, mask=fm)


def _sc_body(pin_h, pout_h,
             t0, a0, v0, d0, e0, f0, t1, a1, v1, d1, e1, f1,
             po0, po1, si0, si1, so0, so1):
    wid = lax.axis_index("s") * NC + lax.axis_index("c")
    lane = lax.iota(jnp.int32, 16)
    m7 = lane < 7
    m2 = lane < 2
    m3 = lane < 3
    secbase = wid * (NSEC * RPW)

    bufs = ((t0, a0, v0, d0, e0, f0), (t1, a1, v1, d1, e1, f1))
    pos = (po0, po1)
    sis = (si0, si1)
    sos = (so0, so1)

    def start_in(c, bsel):
        bt, ba, bv, bd, be, bf = bufs[bsel]
        sem = sis[bsel]
        for s, dst in ((0, bt), (1, ba), (2, bv), (3, bd), (4, be), (5, bf)):
            pltpu.async_copy(
                pin_h.at[pl.ds(secbase + s * RPW + c * CH, CH)], dst, sem)

    def wait_in(bsel):
        for dst in bufs[bsel]:
            pltpu.make_async_copy(pin_h.at[pl.ds(0, CH)], dst,
                                  sis[bsel]).wait()

    def start_out(c, bsel):
        pltpu.async_copy(
            pos[bsel], pout_h.at[pl.ds(wid * RPW + c * CH, CH)], sos[bsel])

    def wait_out(bsel):
        pltpu.make_async_copy(pout_h.at[pl.ds(0, CH)], pos[bsel],
                              sos[bsel]).wait()

    def compute_chunk(bsel):
        sct, sca, scv, xd, xe, xf = bufs[bsel]
        po = pos[bsel]

        @plsc.parallel_loop(0, CH, unroll=2)
        def _(r):
            _branch_row(sct, xd, 16, po, r, 5, m7, lane, F_T, I_T, W_T, m7)
            _branch_row(sca, xe, 0, po, r, 5, m7, lane, F_A, I_A, W_A, m7)
            _branch_row(scv, xf, 0, po, r, 5, m7, lane, F_V, I_V, W_V, m7)
            _branch_row(xd, xd, 84, po, r, 1, m2, lane, F_S, I_S, W_S, m3)

    start_in(0, 0)

    def pair_body(c2, _):
        c = 2 * c2
        start_in(c + 1, 1)
        wait_in(0)

        @pl.when(c2 > 0)
        def _():
            wait_out(0)

        compute_chunk(0)
        start_out(c, 0)

        @pl.when(c2 < NCH // 2 - 1)
        def _():
            start_in(c + 2, 0)

        wait_in(1)

        @pl.when(c2 > 0)
        def _():
            wait_out(1)

        compute_chunk(1)
        start_out(c + 1, 1)
        return 0

    lax.fori_loop(0, NCH // 2, pair_body, 0)
    wait_out(0)
    wait_out(1)


def _sc_topk(pin):
    f32 = jnp.float32
    mesh = plsc.VectorSubcoreMesh(core_axis_name="c", subcore_axis_name="s")
    fn = pl.kernel(
        _sc_body,
        out_type=jax.ShapeDtypeStruct((B, POUT), f32),
        mesh=mesh,
        scratch_types=(
            [pltpu.VMEM((CH, POUT), f32) for _ in range(12)]
            + [pltpu.VMEM((CH, POUT), f32), pltpu.VMEM((CH, POUT), f32)]
            + [pltpu.SemaphoreType.DMA] * 4
        ),
        compiler_params=pltpu.CompilerParams(
            needs_layout_passes=False,
            use_tc_tiling_on_sc=False))
    return fn(pin)


def _final_body(p, f1W, f1b, f2w, f2b,
                outT, x1_ref, ti, tw, ai, aw, vi, vw, si, sw):
    pv = p[...]
    feat = pv[:, 0:24]
    x1 = jnp.dot(feat, f1W[...], preferred_element_type=jnp.float32) + f1b[...]
    x1_ref[...] = x1
    xr = jnp.maximum(x1, 0.0)
    ov = jnp.sum(xr * f2w[...], axis=1, keepdims=True) + f2b[...]
    outT[...] = jnp.transpose(ov)

    def tp(cols):
        return jnp.transpose(cols)

    ti[...] = lax.bitcast_convert_type(tp(pv[:, I_T:I_T + 7]), jnp.int32)
    ai[...] = lax.bitcast_convert_type(tp(pv[:, I_A:I_A + 7]), jnp.int32)
    vi[...] = lax.bitcast_convert_type(tp(pv[:, I_V:I_V + 7]), jnp.int32)
    si[...] = lax.bitcast_convert_type(tp(pv[:, I_S:I_S + 2]), jnp.int32)
    tw[...] = tp(pv[:, W_T:W_T + 7])
    aw[...] = tp(pv[:, W_A:W_A + 7])
    vw[...] = tp(pv[:, W_V:W_V + 7])
    sw[...] = tp(pv[:, W_S:W_S + 2])


def _tc_final(pout, f1Wp, f1b, f2w, f2b, R3=1024):
    grid = (B // R3,)
    f32, i32 = jnp.float32, jnp.int32

    def rows_spec(cols):
        return pl.BlockSpec((R3, cols), lambda i: (i, 0))

    def colsT_spec(rows):
        return pl.BlockSpec((rows, R3), lambda i: (0, i))

    def full_spec(arr):
        return pl.BlockSpec(arr.shape, lambda i: (0,) * arr.ndim)

    return pl.pallas_call(
        _final_body, grid=grid,
        in_specs=[rows_spec(POUT), full_spec(f1Wp), full_spec(f1b),
                  full_spec(f2w), full_spec(f2b)],
        out_specs=[colsT_spec(1), rows_spec(256),
                   colsT_spec(7), colsT_spec(7), colsT_spec(7), colsT_spec(7),
                   colsT_spec(7), colsT_spec(7), colsT_spec(2), colsT_spec(2)],
        out_shape=[jax.ShapeDtypeStruct((1, B), f32),
                   jax.ShapeDtypeStruct((B, 256), f32),
                   jax.ShapeDtypeStruct((7, B), i32),
                   jax.ShapeDtypeStruct((7, B), f32),
                   jax.ShapeDtypeStruct((7, B), i32),
                   jax.ShapeDtypeStruct((7, B), f32),
                   jax.ShapeDtypeStruct((7, B), i32),
                   jax.ShapeDtypeStruct((7, B), f32),
                   jax.ShapeDtypeStruct((2, B), i32),
                   jax.ShapeDtypeStruct((2, B), f32)],
    )(pout, f1Wp, f1b, f2w, f2b)


def kernel(thick, area, vol, sub_vol,
           tW1, tb1, tW2, tb2, aW1, ab1, aW2, ab2,
           vW1, vb1, vW2, vb2, sW1, sb1, sW2, sb2,
           f1W, f1b, f2W, f2b):
    f32 = jnp.float32
    pad60 = jnp.full((60,), NEG, f32)

    def padw_t(W1, b1, W2, b2):
        W2p = jnp.pad(W2, ((0, 0), (0, 60)))
        b2p = jnp.concatenate([b2, pad60])
        return (W1.T, b1.reshape(-1, 1), W2p.T, b2p.reshape(-1, 1))

    tw = padw_t(tW1, tb1, tW2, tb2)
    aw = padw_t(aW1, ab1, aW2, ab2)
    vw = padw_t(vW1, vb1, vW2, vb2)
    sw = (sW1.T, sb1.reshape(-1, 1), sW2.T, sb2.reshape(-1, 1))

    pin = _tc_scores(thick.T, area.T, vol.T, sub_vol.T, tw, aw, vw, sw)
    pout = _sc_topk(pin)

    f1Wp = jnp.pad(f1W, ((0, 1), (0, 0)))
    (outT, x1, tiT, twT, aiT, awT, viT, vwT, siT, swT) = _tc_final(
        pout, f1Wp, f1b.reshape(1, 256), f2W.reshape(1, 256),
        f2b.reshape(1, 1))

    return (outT.T, x1, tiT.T, twT.T, aiT.T, awT.T, viT.T, vwT.T,
            siT.T, swT.T)
